# baseline (speedup 1.0000x reference)
import functools

import jax
import jax.numpy as jnp
from jax import lax
from jax.experimental import pallas as pl
from jax.experimental.pallas import tpu as pltpu
from jax.experimental.pallas import tpu_sc as plsc

N = 10000
D = 128
NC, NS = 2, 16
NW = NC * NS
NPAD = 10240
RZ = NPAD // NS


RING = 2
IR = 4


def _chunking(E):
    per_w = E // NW
    assert per_w * NW == E, E
    for c in range(128, 0, -1):
        if per_w % c == 0 and (per_w // c) % IR == 0:
            return per_w // c, c
    raise AssertionError(E)


def _seg_body(nch, with_cnt, *refs):
    if with_cnt:
        (x_hbm, srcw, dstw, zacc, ones_hbm, accp, cntp,
         acc_sh, *rest) = refs
    else:
        x_hbm, srcw, dstw, zacc, accp, acc_sh, *rest = refs
    src_v = rest[:IR]
    dst_v = rest[IR:2 * IR]
    rows_v = rest[2 * IR:2 * IR + RING]
    gsems = rest[2 * IR + RING:2 * IR + 2 * RING]
    ssems = rest[2 * IR + 2 * RING:2 * IR + 3 * RING]
    isems = rest[2 * IR + 3 * RING:2 * IR + 3 * RING + IR]
    c = lax.axis_index("c")
    s = lax.axis_index("s")
    wid = s * NC + c
    base = wid * nch

    pltpu.sync_copy(zacc.at[pl.ds(s * RZ, RZ)], acc_sh.at[pl.ds(s * RZ, RZ)])
    plsc.subcore_barrier()

    for q in range(IR):
        pltpu.async_copy(srcw.at[base + q], src_v[q], isems[q])
        pltpu.async_copy(dstw.at[base + q], dst_v[q], isems[q])
    for b in range(RING):
        pltpu.make_async_copy(srcw.at[base + b], src_v[b],
                              isems[b]).wait()
        pltpu.make_async_copy(dstw.at[base + b], dst_v[b],
                              isems[b]).wait()
        pltpu.async_copy(x_hbm.at[src_v[b].at[0]], rows_v[b], gsems[b])
    ngrp = nch // IR

    def grp(p, carry):
        for q in range(IR):
            j = IR * p + q
            b = q % RING
            pltpu.make_async_copy(x_hbm.at[src_v[q].at[0]],
                                  rows_v[b], gsems[b]).wait()
            pltpu.sync_copy(rows_v[b], acc_sh.at[dst_v[q].at[0]],
                            add=True)

            @pl.when(j + IR < nch)
            def _():
                pltpu.async_copy(srcw.at[base + j + IR], src_v[q],
                                 isems[q])
                pltpu.async_copy(dstw.at[base + j + IR], dst_v[q],
                                 isems[q])

            @pl.when(j + RING < nch)
            def _():
                q2 = (q + RING) % IR
                pltpu.make_async_copy(srcw.at[base + j + RING],
                                      src_v[q2], isems[q2]).wait()
                pltpu.make_async_copy(dstw.at[base + j + RING],
                                      dst_v[q2], isems[q2]).wait()
                pltpu.async_copy(x_hbm.at[src_v[q2].at[0]],
                                 rows_v[b], gsems[b])
        return carry

    lax.fori_loop(0, ngrp, grp, 0)
    plsc.subcore_barrier()

    pltpu.sync_copy(acc_sh.at[pl.ds(s * RZ, RZ)],
                    accp.at[c, pl.ds(s * RZ, RZ)])

    if with_cnt:
        plsc.subcore_barrier()
        pltpu.sync_copy(zacc.at[pl.ds(s * RZ, RZ)],
                        acc_sh.at[pl.ds(s * RZ, RZ)])
        ones_v = rows_v[0]
        pltpu.sync_copy(ones_hbm, ones_v)
        plsc.subcore_barrier()
        for b in (0, 1):
            pltpu.async_copy(dstw.at[base + b], dst_v[b], isems[b])

        def cpair(p, carry):
            for b in (0, 1):
                j = 2 * p + b
                pltpu.make_async_copy(dstw.at[base + j],
                                      dst_v[b], isems[b]).wait()
                pltpu.sync_copy(ones_v, acc_sh.at[dst_v[b].at[0]], add=True)

                @pl.when(j + 2 < nch)
                def _():
                    pltpu.async_copy(dstw.at[base + j + 2], dst_v[b],
                                     isems[b])
            return carry

        lax.fori_loop(0, nch // 2, cpair, 0)
        plsc.subcore_barrier()
        pltpu.sync_copy(acc_sh.at[pl.ds(s * RZ, RZ)],
                        cntp.at[c, pl.ds(s * RZ, RZ)])


def _make_seg(nch, cs, with_cnt):
    mesh = plsc.VectorSubcoreMesh(core_axis_name="c", subcore_axis_name="s",
                                  num_cores=NC, num_subcores=NS)
    f32 = jnp.float32
    one_out = jax.ShapeDtypeStruct((NC, NPAD, D), f32)
    out_type = (one_out, one_out) if with_cnt else one_out
    scratch = [pltpu.VMEM_SHARED((NPAD, D), f32)]
    scratch += (
        [pltpu.VMEM((1, cs), jnp.int32)] * IR
        + [pltpu.VMEM((1, cs), jnp.int32)] * IR
        + [pltpu.VMEM((cs, D), f32)] * RING
        + [pltpu.SemaphoreType.DMA] * RING
        + [pltpu.SemaphoreType.DMA] * RING
        + [pltpu.SemaphoreType.DMA] * IR
    )
    return pl.kernel(functools.partial(_seg_body, nch, with_cnt),
                     out_type=out_type, mesh=mesh, scratch_types=scratch)


_TC_R = 2000


def _tc_root_body(x, w, b, out_ref):
    out_ref[...] = jnp.dot(x[...], w[...],
                           preferred_element_type=jnp.float32) + b[...]


def _tc_root(x, w_r, b):
    R = _TC_R
    return pl.pallas_call(
        _tc_root_body,
        grid=(N // R,),
        in_specs=[
            pl.BlockSpec((R, D), lambda i: (i, 0)),
            pl.BlockSpec((D, D), lambda i: (0, 0)),
            pl.BlockSpec((1, D), lambda i: (0, 0)),
        ],
        out_specs=pl.BlockSpec((R, D), lambda i: (i, 0)),
        out_shape=jax.ShapeDtypeStruct((N, D), jnp.float32),
    )(x, w_r, b)


def _tc_mean_body(relu, accp, cntp, r, w, out_ref):
    a = accp[0] + accp[1]
    cnt = cntp[0, :, 0:1] + cntp[1, :, 0:1]
    mean = a * (1.0 / jnp.maximum(cnt, 1.0))
    o = jnp.dot(mean, w[...], preferred_element_type=jnp.float32) + r[...]
    if relu:
        o = jnp.maximum(o, 0.0)
    out_ref[...] = o


def _tc_mean(accp, cntp, r, w_l, relu):
    R = _TC_R
    return pl.pallas_call(
        functools.partial(_tc_mean_body, relu),
        grid=(N // R,),
        in_specs=[
            pl.BlockSpec((NC, R, D), lambda i: (0, i, 0)),
            pl.BlockSpec((NC, R, D), lambda i: (0, i, 0)),
            pl.BlockSpec((R, D), lambda i: (i, 0)),
            pl.BlockSpec((D, D), lambda i: (0, 0)),
        ],
        out_specs=pl.BlockSpec((R, D), lambda i: (i, 0)),
        out_shape=jax.ShapeDtypeStruct((N, D), jnp.float32),
    )(accp, cntp, r, w_l)


def kernel(x, edge_index, W1_l, b1_l, W1_r, W2_l, b2_l, W2_r):
    E = edge_index.shape[1]
    nch, cs = _chunking(E)
    ei = edge_index.astype(jnp.int32)
    srcw = ei[0].reshape(NW * nch, 1, cs)
    dstw = ei[1].reshape(NW * nch, 1, cs)
    f32 = jnp.float32
    zacc = jnp.zeros((NPAD, D), f32)
    ones = jnp.ones((cs, D), f32)

    seg1 = _make_seg(nch, cs, with_cnt=True)
    seg = _make_seg(nch, cs, with_cnt=False)

    r1 = _tc_root(x, W1_r, b1_l.reshape(1, D))
    accp1, cntp = seg1(x, srcw, dstw, zacc, ones)
    h = _tc_mean(accp1, cntp, r1, W1_l, relu=True)
    r2 = _tc_root(h, W2_r, b2_l.reshape(1, D))
    accp2 = seg(h, srcw, dstw, zacc)
    out = _tc_mean(accp2, cntp, r2, W2_l, relu=False)
    return out


# speedup vs baseline: 1.0049x; 1.0049x over previous
import functools

import jax
import jax.numpy as jnp
from jax import lax
from jax.experimental import pallas as pl
from jax.experimental.pallas import tpu as pltpu
from jax.experimental.pallas import tpu_sc as plsc

N = 10000
D = 128
NC, NS = 2, 16
NW = NC * NS
NPAD = 10240
RZ = NPAD // NS


RING = 2
IR = 4


def _chunking(E):
    per_w = E // NW
    assert per_w * NW == E, E
    for c in range(128, 0, -1):
        if per_w % c == 0 and (per_w // c) % IR == 0:
            return per_w // c, c
    raise AssertionError(E)


def _seg_body(nch, with_cnt, *refs):
    if with_cnt:
        (x_hbm, srcw, dstw, zacc, ones_hbm, accp, cntp,
         acc_sh, *rest) = refs
    else:
        x_hbm, srcw, dstw, zacc, accp, acc_sh, *rest = refs
    src_v = rest[:IR]
    dst_v = rest[IR:2 * IR]
    rows_v = rest[2 * IR:2 * IR + RING]
    gsems = rest[2 * IR + RING:2 * IR + 2 * RING]
    ssems = rest[2 * IR + 2 * RING:2 * IR + 3 * RING]
    isems = rest[2 * IR + 3 * RING:2 * IR + 3 * RING + IR]
    c = lax.axis_index("c")
    s = lax.axis_index("s")
    wid = s * NC + c
    base = wid * nch

    pltpu.sync_copy(zacc.at[pl.ds(s * RZ, RZ)], acc_sh.at[pl.ds(s * RZ, RZ)])
    plsc.subcore_barrier()

    for q in range(IR):
        pltpu.async_copy(srcw.at[base + q], src_v[q], isems[q])
        pltpu.async_copy(dstw.at[base + q], dst_v[q], isems[q])
    for b in range(RING):
        pltpu.make_async_copy(srcw.at[base + b], src_v[b],
                              isems[b]).wait()
        pltpu.make_async_copy(dstw.at[base + b], dst_v[b],
                              isems[b]).wait()
        pltpu.async_copy(x_hbm.at[src_v[b].at[0]], rows_v[b], gsems[b])
    ngrp = nch // IR

    def grp(p, carry):
        for q in range(IR):
            j = IR * p + q
            b = q % RING
            pltpu.make_async_copy(x_hbm.at[src_v[q].at[0]],
                                  rows_v[b], gsems[b]).wait()
            pltpu.sync_copy(rows_v[b], acc_sh.at[dst_v[q].at[0]],
                            add=True)

            @pl.when(j + IR < nch)
            def _():
                pltpu.async_copy(srcw.at[base + j + IR], src_v[q],
                                 isems[q])
                pltpu.async_copy(dstw.at[base + j + IR], dst_v[q],
                                 isems[q])

            @pl.when(j + RING < nch)
            def _():
                q2 = (q + RING) % IR
                pltpu.make_async_copy(srcw.at[base + j + RING],
                                      src_v[q2], isems[q2]).wait()
                pltpu.make_async_copy(dstw.at[base + j + RING],
                                      dst_v[q2], isems[q2]).wait()
                pltpu.async_copy(x_hbm.at[src_v[q2].at[0]],
                                 rows_v[b], gsems[b])
        return carry

    lax.fori_loop(0, ngrp, grp, 0)
    plsc.subcore_barrier()

    pltpu.sync_copy(acc_sh.at[pl.ds(s * RZ, RZ)],
                    accp.at[c, pl.ds(s * RZ, RZ)])

    if with_cnt:
        plsc.subcore_barrier()
        pltpu.sync_copy(zacc.at[pl.ds(s * RZ, RZ)],
                        acc_sh.at[pl.ds(s * RZ, RZ)])
        ones_v = rows_v[0]
        pltpu.sync_copy(ones_hbm, ones_v)
        plsc.subcore_barrier()
        for b in (0, 1):
            pltpu.async_copy(dstw.at[base + b], dst_v[b], isems[b])

        def cpair(p, carry):
            for b in (0, 1):
                j = 2 * p + b
                pltpu.make_async_copy(dstw.at[base + j],
                                      dst_v[b], isems[b]).wait()
                pltpu.sync_copy(ones_v, acc_sh.at[dst_v[b].at[0]], add=True)

                @pl.when(j + 2 < nch)
                def _():
                    pltpu.async_copy(dstw.at[base + j + 2], dst_v[b],
                                     isems[b])
            return carry

        lax.fori_loop(0, nch // 2, cpair, 0)
        plsc.subcore_barrier()
        pltpu.sync_copy(acc_sh.at[pl.ds(s * RZ, RZ)],
                        cntp.at[c, pl.ds(s * RZ, RZ)])


def _make_seg(nch, cs, with_cnt):
    mesh = plsc.VectorSubcoreMesh(core_axis_name="c", subcore_axis_name="s",
                                  num_cores=NC, num_subcores=NS)
    f32 = jnp.float32
    one_out = jax.ShapeDtypeStruct((NC, NPAD, D), f32)
    out_type = (one_out, one_out) if with_cnt else one_out
    scratch = [pltpu.VMEM_SHARED((NPAD, D), f32)]
    scratch += (
        [pltpu.VMEM((1, cs), jnp.int32)] * IR
        + [pltpu.VMEM((1, cs), jnp.int32)] * IR
        + [pltpu.VMEM((cs, D), f32)] * RING
        + [pltpu.SemaphoreType.DMA] * RING
        + [pltpu.SemaphoreType.DMA] * RING
        + [pltpu.SemaphoreType.DMA] * IR
    )
    return pl.kernel(functools.partial(_seg_body, nch, with_cnt),
                     out_type=out_type, mesh=mesh, scratch_types=scratch)


def _tc_body(relu, accp, cntp, x, w, b, out_ref):
    a = accp[0] + accp[1]
    cnt = cntp[0, :, 0:1] + cntp[1, :, 0:1]
    mean = a * (1.0 / jnp.maximum(cnt, 1.0))
    cat = jnp.concatenate([mean, x[...]], axis=1)
    o = jnp.dot(cat, w[...], preferred_element_type=jnp.float32) + b[...]
    if relu:
        o = jnp.maximum(o, 0.0)
    out_ref[...] = o


def _tc_layer(accp, cntp, x, wcat, b, relu):
    R = 2000
    grid = (N // R,)
    return pl.pallas_call(
        functools.partial(_tc_body, relu),
        grid=grid,
        in_specs=[
            pl.BlockSpec((NC, R, D), lambda i: (0, i, 0)),
            pl.BlockSpec((NC, R, D), lambda i: (0, i, 0)),
            pl.BlockSpec((R, D), lambda i: (i, 0)),
            pl.BlockSpec((2 * D, D), lambda i: (0, 0)),
            pl.BlockSpec((1, D), lambda i: (0, 0)),
        ],
        out_specs=pl.BlockSpec((R, D), lambda i: (i, 0)),
        out_shape=jax.ShapeDtypeStruct((N, D), jnp.float32),
    )(accp, cntp, x, wcat, b)


def kernel(x, edge_index, W1_l, b1_l, W1_r, W2_l, b2_l, W2_r):
    E = edge_index.shape[1]
    nch, cs = _chunking(E)
    ei = edge_index.astype(jnp.int32)
    srcw = ei[0].reshape(NW * nch, 1, cs)
    dstw = ei[1].reshape(NW * nch, 1, cs)
    f32 = jnp.float32
    zacc = jnp.zeros((NPAD, D), f32)
    ones = jnp.ones((cs, D), f32)

    seg1 = _make_seg(nch, cs, with_cnt=True)
    seg = _make_seg(nch, cs, with_cnt=False)

    accp1, cntp = seg1(x, srcw, dstw, zacc, ones)
    h = _tc_layer(accp1, cntp, x,
                  jnp.concatenate([W1_l, W1_r], axis=0),
                  b1_l.reshape(1, D), relu=True)
    accp2 = seg(h, srcw, dstw, zacc)
    out = _tc_layer(accp2, cntp, h,
                    jnp.concatenate([W2_l, W2_r], axis=0),
                    b2_l.reshape(1, D), relu=False)
    return out
